# baseline (speedup 1.0000x reference)
import functools
import jax
import jax.numpy as jnp
from jax import lax
from jax.experimental import pallas as pl
from jax.experimental.pallas import tpu as pltpu
from jax.experimental.pallas import tpu_sc as plsc

N_NODES = 10000
N_EDGES = 320000
DIM = 128
NC, NS = 2, 16
NW = NC * NS
EPT = 2 * N_EDGES // NW
CH = 80
NCHUNK = EPT // CH
WR_TILES = 10
WR_ROWS = N_NODES // WR_TILES
WCH = 40
BE = 2560
NBLK_E = 2 * N_EDGES // BE
NBLK_SIDE = N_EDGES // BE
BN = 1000
NBLK_N = N_NODES // BN

f32 = jnp.float32


GB_G = 5
NGRP_G = NCHUNK // GB_G


def _gather_body(table, idxg, out, ib, rb, gsem, isem0, isem1, wsem0, wsem1):
    cid = lax.axis_index("c")
    sid = lax.axis_index("s")
    w = cid * NS + sid
    isem = (isem0, isem1)
    wsem = (wsem0, wsem1)

    pltpu.async_copy(idxg.at[w].at[0], ib.at[0], isem[0])

    def group(g, b):
        @pl.when(g + 1 < NGRP_G)
        def _pf():
            pltpu.async_copy(idxg.at[w].at[g + 1], ib.at[1 - b], isem[1 - b])

        @pl.when(g >= 2)
        def _dw():
            for k in range(GB_G):
                pltpu.make_async_copy(
                    rb.at[b].at[k],
                    out.at[cid].at[pl.ds(sid * EPT, CH)], wsem[b]).wait()

        pltpu.make_async_copy(idxg.at[w].at[g], ib.at[b], isem[b]).wait()

        descs = []
        for k in range(GB_G):
            descs.append(pltpu.async_copy(
                table.at[ib.at[b].at[k]],
                rb.at[b].at[k], gsem))
        for d in descs:
            d.wait()

        for k in range(GB_G):
            pltpu.async_copy(
                rb.at[b].at[k],
                out.at[cid].at[pl.ds(sid * EPT + (g * GB_G + k) * CH, CH)],
                wsem[b])

    def pair(i, _):
        group(2 * i, 0)
        group(2 * i + 1, 1)
        return 0

    lax.fori_loop(0, NGRP_G // 2, pair, 0)

    for b in range(2):
        for k in range(GB_G):
            pltpu.make_async_copy(
                rb.at[b].at[k],
                out.at[cid].at[pl.ds(sid * EPT, CH)], wsem[b]).wait()


@functools.cache
def _sc_mesh():
    return plsc.VectorSubcoreMesh(
        core_axis_name="c", subcore_axis_name="s",
        num_cores=NC, num_subcores=NS)


@functools.cache
def _sc_gather_kernel():
    return pl.kernel(
        _gather_body,
        out_type=jax.ShapeDtypeStruct((2, N_EDGES, DIM), f32),
        mesh=_sc_mesh(),
        scratch_types=[
            pltpu.VMEM((2, GB_G, CH), jnp.int32),
            pltpu.VMEM((2, GB_G, CH, DIM), f32),
            pltpu.SemaphoreType.DMA,
            pltpu.SemaphoreType.DMA,
            pltpu.SemaphoreType.DMA,
            pltpu.SemaphoreType.DMA,
            pltpu.SemaphoreType.DMA,
        ],
    )


def _sc_gather(table, idxg):
    return _sc_gather_kernel()(table, idxg)


def _maybe_when(cond, fn):
    if isinstance(cond, bool):
        if cond:
            fn()
    else:
        pl.when(cond)(fn)


def _zero_acc(sid, stage, acc):
    row0 = sid * WR_ROWS
    zrow = jnp.zeros((16,), f32)

    @pl.when(sid < WR_TILES)
    def _zero():
        def z0(i, _):
            def z1(k, _):
                stage[i, pl.ds(k * 16, 16)] = zrow
                return 0
            lax.fori_loop(0, DIM // 16, z1, 0)
            return 0
        lax.fori_loop(0, WCH, z0, 0)
        for k in range(WR_ROWS // WCH):
            pltpu.sync_copy(stage, acc.at[pl.ds(row0 + k * WCH, WCH)])


def _write_acc(cid, sid, stage, acc, out):
    row0 = sid * WR_ROWS

    @pl.when(sid < WR_TILES)
    def _writeout():
        for k in range(WR_ROWS // WCH):
            r = row0 + k * WCH
            pltpu.sync_copy(acc.at[pl.ds(r, WCH)], stage)
            pltpu.sync_copy(stage, out.at[cid].at[pl.ds(r, WCH)])


GB_S = 2
NGRP_S = NCHUNK // GB_S


def _scatter_body(svals, idxs, out_sum,
                  ib, vb, acc, isem0, isem1, vsem0, vsem1, ssem0, ssem1):
    cid = lax.axis_index("c")
    sid = lax.axis_index("s")
    w = cid * NS + sid
    isem = (isem0, isem1)
    vsem = (vsem0, vsem1)
    ssem = (ssem0, ssem1)

    _zero_acc(sid, vb.at[0].at[0].at[pl.ds(0, WCH)], acc)
    plsc.subcore_barrier()

    def fire_loads(g, b):
        for k in range(GB_S):
            j = g * GB_S + k
            pltpu.async_copy(idxs.at[w].at[j], ib.at[b].at[k], isem[b])
            pltpu.async_copy(
                svals.at[cid].at[pl.ds(sid * EPT + j * CH, CH)],
                vb.at[b].at[k], vsem[b])

    def group(g, b):
        def _da():
            for k in range(GB_S):
                pltpu.make_async_copy(
                    vb.at[1 - b].at[k], acc.at[ib.at[1 - b].at[k]],
                    ssem[1 - b]).wait()
        _maybe_when(g >= 1, _da)

        _maybe_when(g + 1 < NGRP_S, lambda: fire_loads(g + 1, 1 - b))

        for k in range(GB_S):
            pltpu.make_async_copy(
                idxs.at[w].at[0], ib.at[b].at[k], isem[b]).wait()
            pltpu.make_async_copy(
                svals.at[cid].at[pl.ds(sid * EPT, CH)], vb.at[b].at[k],
                vsem[b]).wait()

        for k in range(GB_S):
            pltpu.async_copy(vb.at[b].at[k], acc.at[ib.at[b].at[k]],
                             ssem[b], add=True)

    fire_loads(0, 0)

    def pair(i, _):
        group(2 * i, 0)
        group(2 * i + 1, 1)
        return 0

    lax.fori_loop(0, (NGRP_S - 1) // 2, pair, 0)
    group(NGRP_S - 1, 0)

    for k in range(GB_S):
        pltpu.make_async_copy(
            vb.at[0].at[k], acc.at[ib.at[0].at[k]], ssem[0]).wait()

    plsc.subcore_barrier()
    _write_acc(cid, sid, vb.at[0].at[0].at[pl.ds(0, WCH)], acc, out_sum)


def _count_body(idx3d, out_cnt, idx_v, ones_v, stage, acc):
    cid = lax.axis_index("c")
    sid = lax.axis_index("s")
    w = cid * NS + sid
    one_row = jnp.ones((16,), f32)

    _zero_acc(sid, stage, acc)

    def o0(i, _):
        def o1(k, _):
            ones_v[i, pl.ds(k * 16, 16)] = one_row
            return 0
        lax.fori_loop(0, DIM // 16, o1, 0)
        return 0
    lax.fori_loop(0, CH, o0, 0)

    plsc.subcore_barrier()

    def step(j, _):
        pltpu.sync_copy(idx3d.at[w].at[j], idx_v)
        pltpu.sync_copy(ones_v, acc.at[idx_v], add=True)
        return 0

    lax.fori_loop(0, NCHUNK, step, 0)

    plsc.subcore_barrier()
    _write_acc(cid, sid, stage, acc, out_cnt)


@functools.cache
def _sc_scatter_kernel():
    return pl.kernel(
        _scatter_body,
        out_type=jax.ShapeDtypeStruct((2, N_NODES, DIM), f32),
        mesh=_sc_mesh(),
        scratch_types=[
            pltpu.VMEM((2, GB_S, CH), jnp.int32),
            pltpu.VMEM((2, GB_S, CH, DIM), f32),
            pltpu.VMEM_SHARED((N_NODES, DIM), f32),
            pltpu.SemaphoreType.DMA,
            pltpu.SemaphoreType.DMA,
            pltpu.SemaphoreType.DMA,
            pltpu.SemaphoreType.DMA,
            pltpu.SemaphoreType.DMA,
            pltpu.SemaphoreType.DMA,
        ],
    )


def _sc_scatter(svals, idxs):
    return _sc_scatter_kernel()(svals, idxs)


@functools.cache
def _sc_count_kernel():
    return pl.kernel(
        _count_body,
        out_type=jax.ShapeDtypeStruct((2, N_NODES, DIM), f32),
        mesh=_sc_mesh(),
        scratch_types=[
            pltpu.VMEM((CH,), jnp.int32),
            pltpu.VMEM((CH, DIM), f32),
            pltpu.VMEM((WCH, DIM), f32),
            pltpu.VMEM_SHARED((N_NODES, DIM), f32),
        ],
    )


def _sc_count(idx3d):
    return _sc_count_kernel()(idx3d)


def _node_mlp_body(x, w1, b1, w2, b2, o):
    h = jnp.maximum(jnp.dot(x[...], w1[...], preferred_element_type=f32)
                    + b1[...], 0.0)
    o[...] = jnp.maximum(jnp.dot(h, w2[...], preferred_element_type=f32)
                         + b2[...], 0.0)


def _node_mlp(x, w1, b1, w2, b2):
    return pl.pallas_call(
        _node_mlp_body,
        grid=(NBLK_N,),
        in_specs=[
            pl.BlockSpec((BN, DIM), lambda i: (i, 0)),
            pl.BlockSpec((DIM, 256), lambda i: (0, 0)),
            pl.BlockSpec((1, 256), lambda i: (0, 0)),
            pl.BlockSpec((256, DIM), lambda i: (0, 0)),
            pl.BlockSpec((1, DIM), lambda i: (0, 0)),
        ],
        out_specs=pl.BlockSpec((BN, DIM), lambda i: (i, 0)),
        out_shape=jax.ShapeDtypeStruct((N_NODES, DIM), f32),
        compiler_params=pltpu.CompilerParams(
            dimension_semantics=("parallel",)),
    )(x, w1, b1, w2, b2)


def _edge_feat_body(x, w1, b1, w2, b2, o):
    h = jnp.maximum(x[...][:, None] * w1[...] + b1[...], 0.0)
    o[...] = jnp.maximum(
        jnp.dot(h.astype(jnp.bfloat16), w2[...],
                preferred_element_type=f32) + b2[...],
        0.0).astype(jnp.bfloat16)


BE_F = 512


def _edge_feat(x, w1, b1, w2, b2):
    return pl.pallas_call(
        _edge_feat_body,
        grid=(2 * N_EDGES // BE_F,),
        in_specs=[
            pl.BlockSpec((BE_F,), lambda i: (i,)),
            pl.BlockSpec((1, 256), lambda i: (0, 0)),
            pl.BlockSpec((1, 256), lambda i: (0, 0)),
            pl.BlockSpec((256, DIM), lambda i: (0, 0)),
            pl.BlockSpec((1, DIM), lambda i: (0, 0)),
        ],
        out_specs=pl.BlockSpec((BE_F, DIM), lambda i: (i, 0)),
        out_shape=jax.ShapeDtypeStruct((2 * N_EDGES, DIM), jnp.bfloat16),
        compiler_params=pltpu.CompilerParams(
            dimension_semantics=("parallel",)),
    )(x, w1, b1, w2, b2)


def _msg_body(xp, xc, fp, fc, w1ap, w1bp, w1cp, b1p, w2p, b2p,
              w1ac, w1bc, w1cc, b1c, w2c, b2c, o):
    bf = jnp.bfloat16
    xpb = xp[...][0].astype(bf)
    xcb = xc[...][0].astype(bf)
    hp = (jnp.dot(xcb, w1ap[...], preferred_element_type=f32)
          + jnp.dot(xpb, w1bp[...], preferred_element_type=f32)
          + jnp.dot(fp[...], w1cp[...], preferred_element_type=f32)
          + b1p[...])
    hp = jnp.maximum(hp, 0.0)
    o[0] = jnp.maximum(
        jnp.dot(hp.astype(bf), w2p[...], preferred_element_type=f32)
        + b2p[...], 0.0)
    hc = (jnp.dot(xpb, w1ac[...], preferred_element_type=f32)
          + jnp.dot(xcb, w1bc[...], preferred_element_type=f32)
          + jnp.dot(fc[...], w1cc[...], preferred_element_type=f32)
          + b1c[...])
    hc = jnp.maximum(hc, 0.0)
    o[1] = jnp.maximum(
        jnp.dot(hc.astype(bf), w2c[...], preferred_element_type=f32)
        + b2c[...], 0.0)


def _msg_mlp(b_all, f_all, wp, wc):
    wspec = pl.BlockSpec((DIM, 256), lambda i: (0, 0))
    bspec = pl.BlockSpec((1, 256), lambda i: (0, 0))
    w2spec = pl.BlockSpec((256, DIM), lambda i: (0, 0))
    b2spec = pl.BlockSpec((1, DIM), lambda i: (0, 0))
    return pl.pallas_call(
        _msg_body,
        grid=(NBLK_SIDE,),
        in_specs=[
            pl.BlockSpec((1, BE, DIM), lambda i: (0, i, 0)),
            pl.BlockSpec((1, BE, DIM), lambda i: (1, i, 0)),
            pl.BlockSpec((BE, DIM), lambda i: (i, 0)),
            pl.BlockSpec((BE, DIM), lambda i: (NBLK_SIDE + i, 0)),
            wspec, wspec, wspec, bspec, w2spec, b2spec,
            wspec, wspec, wspec, bspec, w2spec, b2spec,
        ],
        out_specs=pl.BlockSpec((2, BE, DIM), lambda i: (0, i, 0)),
        out_shape=jax.ShapeDtypeStruct((2, N_EDGES, DIM), f32),
        compiler_params=pltpu.CompilerParams(
            dimension_semantics=("arbitrary",)),
    )(b_all, b_all, f_all, f_all, *wp, *wc)


def _combine_body(h_ref, sp, ip, mp, sc_, ic, mc, wa, wb, wc, b1, w2, b2, o):
    h = h_ref[...]
    s_p = sp[...][0] * ip[...][0] + mp[...]
    s_c = sc_[...][0] * ic[...][0] + mc[...]
    z = (jnp.dot(h, wa[...], preferred_element_type=f32)
         + jnp.dot(s_p, wb[...], preferred_element_type=f32)
         + jnp.dot(s_c, wc[...], preferred_element_type=f32)
         + b1[...])
    z = jnp.maximum(z, 0.0)
    z = jnp.maximum(jnp.dot(z, w2[...], preferred_element_type=f32)
                    + b2[...], 0.0)
    o[...] = jnp.maximum(h + z, 0.0)


def _combine(hidden, sums, inv, mask_p, mask_c, wa, wb, wc, b1, w2, b2):
    return pl.pallas_call(
        _combine_body,
        grid=(NBLK_N,),
        in_specs=[
            pl.BlockSpec((BN, DIM), lambda i: (i, 0)),
            pl.BlockSpec((1, BN, DIM), lambda i: (0, i, 0)),
            pl.BlockSpec((1, BN, 1), lambda i: (0, i, 0)),
            pl.BlockSpec((BN, DIM), lambda i: (i, 0)),
            pl.BlockSpec((1, BN, DIM), lambda i: (1, i, 0)),
            pl.BlockSpec((1, BN, 1), lambda i: (1, i, 0)),
            pl.BlockSpec((BN, DIM), lambda i: (i, 0)),
            pl.BlockSpec((DIM, 256), lambda i: (0, 0)),
            pl.BlockSpec((DIM, 256), lambda i: (0, 0)),
            pl.BlockSpec((DIM, 256), lambda i: (0, 0)),
            pl.BlockSpec((1, 256), lambda i: (0, 0)),
            pl.BlockSpec((256, DIM), lambda i: (0, 0)),
            pl.BlockSpec((1, DIM), lambda i: (0, 0)),
        ],
        out_specs=pl.BlockSpec((BN, DIM), lambda i: (i, 0)),
        out_shape=jax.ShapeDtypeStruct((N_NODES, DIM), f32),
        compiler_params=pltpu.CompilerParams(
            dimension_semantics=("parallel",)),
    )(hidden, sums, inv, mask_p, sums, inv, mask_c, wa, wb, wc, b1, w2, b2)


def kernel(batch_token, edge_p_node, edge_c_node, edge_p_indicate,
           edge_c_indicate, p_mask, c_mask, start_token, end_token,
           Wv1, bv1, Wv2, bv2, We1, be1, We2, be2, Wp1, bp1, Wp2, bp2,
           Wc1, bc1, Wc2, bc2, Wa1, ba1, Wa2, ba2):
    idx_all = jnp.concatenate([edge_p_node, edge_c_node]).astype(jnp.int32)
    idxg = idx_all.reshape(NW, NGRP_G, GB_G, CH)
    idxc = idx_all.reshape(NW, NCHUNK, CH)
    x_all = jnp.concatenate([edge_p_indicate, edge_c_indicate])
    mask_p = p_mask[:, None] * start_token[None, :]
    mask_c = c_mask[:, None] * end_token[None, :]

    hidden = _node_mlp(batch_token, Wv1, bv1.reshape(1, -1),
                       Wv2, bv2.reshape(1, -1))
    f_all = _edge_feat(x_all, We1, be1.reshape(1, -1),
                       We2.astype(jnp.bfloat16), be2.reshape(1, -1))

    bf16 = jnp.bfloat16
    wp = (Wp1[0:DIM].astype(bf16), Wp1[DIM:2 * DIM].astype(bf16),
          Wp1[2 * DIM:].astype(bf16), bp1.reshape(1, 256),
          Wp2.astype(bf16), bp2.reshape(1, DIM))
    wc = (Wc1[0:DIM].astype(bf16), Wc1[DIM:2 * DIM].astype(bf16),
          Wc1[2 * DIM:].astype(bf16), bc1.reshape(1, 256),
          Wc2.astype(bf16), bc2.reshape(1, DIM))

    wagg_a = Wa1[0:DIM]
    wagg_b = Wa1[DIM:2 * DIM]
    wagg_c = Wa1[2 * DIM:]

    cnts = _sc_count(idxc)
    inv = (1.0 / jnp.maximum(cnts[:, :, 0], 1.0)).reshape(2, N_NODES, 1)

    for _ in range(2):
        b_all = _sc_gather(hidden, idxg)
        s_all = _msg_mlp(b_all, f_all, wp, wc)
        sums = _sc_scatter(s_all, idxc)
        hidden = _combine(hidden, sums, inv, mask_p, mask_c,
                          wagg_a, wagg_b, wagg_c, ba1.reshape(1, -1),
                          Wa2, ba2.reshape(1, -1))
    return hidden


# speedup vs baseline: 1.3363x; 1.3363x over previous
import functools
import jax
import jax.numpy as jnp
from jax import lax
from jax.experimental import pallas as pl
from jax.experimental.pallas import tpu as pltpu
from jax.experimental.pallas import tpu_sc as plsc

N_NODES = 10000
N_EDGES = 320000
DIM = 128
NC, NS = 2, 16
NW = NC * NS
EPT = 2 * N_EDGES // NW
CH = 80
NCHUNK = EPT // CH
WR_TILES = 10
WR_ROWS = N_NODES // WR_TILES
WCH = 40
BE = 2560
NBLK_E = 2 * N_EDGES // BE
NBLK_SIDE = N_EDGES // BE
BN = 1000
NBLK_N = N_NODES // BN

f32 = jnp.float32


GB_G = 5
NGRP_G = NCHUNK // GB_G


def _gather_body(table, idxg, out, ib, rb, gsem, isem0, isem1, wsem0, wsem1):
    cid = lax.axis_index("c")
    sid = lax.axis_index("s")
    w = cid * NS + sid
    isem = (isem0, isem1)
    wsem = (wsem0, wsem1)

    pltpu.async_copy(idxg.at[w].at[0], ib.at[0], isem[0])

    def group(g, b):
        @pl.when(g + 1 < NGRP_G)
        def _pf():
            pltpu.async_copy(idxg.at[w].at[g + 1], ib.at[1 - b], isem[1 - b])

        @pl.when(g >= 2)
        def _dw():
            for k in range(GB_G):
                pltpu.make_async_copy(
                    rb.at[b].at[k],
                    out.at[cid].at[pl.ds(sid * EPT, CH)], wsem[b]).wait()

        pltpu.make_async_copy(idxg.at[w].at[g], ib.at[b], isem[b]).wait()

        descs = []
        for k in range(GB_G):
            descs.append(pltpu.async_copy(
                table.at[ib.at[b].at[k]],
                rb.at[b].at[k], gsem))
        for d in descs:
            d.wait()

        for k in range(GB_G):
            pltpu.async_copy(
                rb.at[b].at[k],
                out.at[cid].at[pl.ds(sid * EPT + (g * GB_G + k) * CH, CH)],
                wsem[b])

    def pair(i, _):
        group(2 * i, 0)
        group(2 * i + 1, 1)
        return 0

    lax.fori_loop(0, NGRP_G // 2, pair, 0)

    for b in range(2):
        for k in range(GB_G):
            pltpu.make_async_copy(
                rb.at[b].at[k],
                out.at[cid].at[pl.ds(sid * EPT, CH)], wsem[b]).wait()


@functools.cache
def _sc_mesh():
    return plsc.VectorSubcoreMesh(
        core_axis_name="c", subcore_axis_name="s",
        num_cores=NC, num_subcores=NS)


@functools.cache
def _sc_gather_kernel():
    return pl.kernel(
        _gather_body,
        out_type=jax.ShapeDtypeStruct((2, N_EDGES, DIM), f32),
        mesh=_sc_mesh(),
        scratch_types=[
            pltpu.VMEM((2, GB_G, CH), jnp.int32),
            pltpu.VMEM((2, GB_G, CH, DIM), f32),
            pltpu.SemaphoreType.DMA,
            pltpu.SemaphoreType.DMA,
            pltpu.SemaphoreType.DMA,
            pltpu.SemaphoreType.DMA,
            pltpu.SemaphoreType.DMA,
        ],
    )


def _sc_gather(table, idxg):
    return _sc_gather_kernel()(table, idxg)


def _maybe_when(cond, fn):
    if isinstance(cond, bool):
        if cond:
            fn()
    else:
        pl.when(cond)(fn)


def _zero_acc(sid, stage, acc):
    row0 = sid * WR_ROWS
    zrow = jnp.zeros((16,), f32)

    @pl.when(sid < WR_TILES)
    def _zero():
        def z0(i, _):
            def z1(k, _):
                stage[i, pl.ds(k * 16, 16)] = zrow
                return 0
            lax.fori_loop(0, DIM // 16, z1, 0)
            return 0
        lax.fori_loop(0, WCH, z0, 0)
        for k in range(WR_ROWS // WCH):
            pltpu.sync_copy(stage, acc.at[pl.ds(row0 + k * WCH, WCH)])


def _write_acc(cid, sid, stage, acc, out):
    row0 = sid * WR_ROWS

    @pl.when(sid < WR_TILES)
    def _writeout():
        for k in range(WR_ROWS // WCH):
            r = row0 + k * WCH
            pltpu.sync_copy(acc.at[pl.ds(r, WCH)], stage)
            pltpu.sync_copy(stage, out.at[cid].at[pl.ds(r, WCH)])


GB_S = 2
NGRP_S = NCHUNK // GB_S


def _scatter_body(svals, idxs, out_sum,
                  ib, vb, acc, isem0, isem1, vsem0, vsem1, ssem0, ssem1):
    cid = lax.axis_index("c")
    sid = lax.axis_index("s")
    w = cid * NS + sid
    isem = (isem0, isem1)
    vsem = (vsem0, vsem1)
    ssem = (ssem0, ssem1)

    _zero_acc(sid, vb.at[0].at[0].at[pl.ds(0, WCH)], acc)
    plsc.subcore_barrier()

    def fire_loads(g, b):
        for k in range(GB_S):
            j = g * GB_S + k
            pltpu.async_copy(idxs.at[w].at[j], ib.at[b].at[k], isem[b])
            pltpu.async_copy(
                svals.at[cid].at[pl.ds(sid * EPT + j * CH, CH)],
                vb.at[b].at[k], vsem[b])

    def group(g, b):
        def _da():
            for k in range(GB_S):
                pltpu.make_async_copy(
                    vb.at[1 - b].at[k], acc.at[ib.at[1 - b].at[k]],
                    ssem[1 - b]).wait()
        _maybe_when(g >= 1, _da)

        _maybe_when(g + 1 < NGRP_S, lambda: fire_loads(g + 1, 1 - b))

        for k in range(GB_S):
            pltpu.make_async_copy(
                idxs.at[w].at[0], ib.at[b].at[k], isem[b]).wait()
            pltpu.make_async_copy(
                svals.at[cid].at[pl.ds(sid * EPT, CH)], vb.at[b].at[k],
                vsem[b]).wait()

        for k in range(GB_S):
            pltpu.async_copy(vb.at[b].at[k], acc.at[ib.at[b].at[k]],
                             ssem[b], add=True)

    fire_loads(0, 0)

    def pair(i, _):
        group(2 * i, 0)
        group(2 * i + 1, 1)
        return 0

    lax.fori_loop(0, (NGRP_S - 1) // 2, pair, 0)
    group(NGRP_S - 1, 0)

    for k in range(GB_S):
        pltpu.make_async_copy(
            vb.at[0].at[k], acc.at[ib.at[0].at[k]], ssem[0]).wait()

    plsc.subcore_barrier()
    _write_acc(cid, sid, vb.at[0].at[0].at[pl.ds(0, WCH)], acc, out_sum)


def _count_body(idx3d, out_cnt, idx_v, ones_v, stage, acc):
    cid = lax.axis_index("c")
    sid = lax.axis_index("s")
    w = cid * NS + sid
    one_row = jnp.ones((16,), f32)

    _zero_acc(sid, stage, acc)

    def o0(i, _):
        def o1(k, _):
            ones_v[i, pl.ds(k * 16, 16)] = one_row
            return 0
        lax.fori_loop(0, DIM // 16, o1, 0)
        return 0
    lax.fori_loop(0, CH, o0, 0)

    plsc.subcore_barrier()

    def step(j, _):
        pltpu.sync_copy(idx3d.at[w].at[j], idx_v)
        pltpu.sync_copy(ones_v, acc.at[idx_v], add=True)
        return 0

    lax.fori_loop(0, NCHUNK, step, 0)

    plsc.subcore_barrier()
    _write_acc(cid, sid, stage, acc, out_cnt)


@functools.cache
def _sc_scatter_kernel():
    return pl.kernel(
        _scatter_body,
        out_type=jax.ShapeDtypeStruct((2, N_NODES, DIM), f32),
        mesh=_sc_mesh(),
        scratch_types=[
            pltpu.VMEM((2, GB_S, CH), jnp.int32),
            pltpu.VMEM((2, GB_S, CH, DIM), f32),
            pltpu.VMEM_SHARED((N_NODES, DIM), f32),
            pltpu.SemaphoreType.DMA,
            pltpu.SemaphoreType.DMA,
            pltpu.SemaphoreType.DMA,
            pltpu.SemaphoreType.DMA,
            pltpu.SemaphoreType.DMA,
            pltpu.SemaphoreType.DMA,
        ],
    )


def _sc_scatter(svals, idxs):
    return _sc_scatter_kernel()(svals, idxs)


@functools.cache
def _sc_count_kernel():
    return pl.kernel(
        _count_body,
        out_type=jax.ShapeDtypeStruct((2, N_NODES, DIM), f32),
        mesh=_sc_mesh(),
        scratch_types=[
            pltpu.VMEM((CH,), jnp.int32),
            pltpu.VMEM((CH, DIM), f32),
            pltpu.VMEM((WCH, DIM), f32),
            pltpu.VMEM_SHARED((N_NODES, DIM), f32),
        ],
    )


def _sc_count(idx3d):
    return _sc_count_kernel()(idx3d)


def _node_mlp_body(x, w1, b1, w2, b2, o):
    h = jnp.maximum(jnp.dot(x[...], w1[...], preferred_element_type=f32)
                    + b1[...], 0.0)
    o[...] = jnp.maximum(jnp.dot(h, w2[...], preferred_element_type=f32)
                         + b2[...], 0.0)


def _node_mlp(x, w1, b1, w2, b2):
    return pl.pallas_call(
        _node_mlp_body,
        grid=(NBLK_N,),
        in_specs=[
            pl.BlockSpec((BN, DIM), lambda i: (i, 0)),
            pl.BlockSpec((DIM, 256), lambda i: (0, 0)),
            pl.BlockSpec((1, 256), lambda i: (0, 0)),
            pl.BlockSpec((256, DIM), lambda i: (0, 0)),
            pl.BlockSpec((1, DIM), lambda i: (0, 0)),
        ],
        out_specs=pl.BlockSpec((BN, DIM), lambda i: (i, 0)),
        out_shape=jax.ShapeDtypeStruct((N_NODES, DIM), f32),
        compiler_params=pltpu.CompilerParams(
            dimension_semantics=("parallel",)),
    )(x, w1, b1, w2, b2)


def _edge_feat_body(x, w1c, b1c, w2t, b2c, o):
    h = jnp.maximum(w1c[...] * x[...] + b1c[...], 0.0)
    o[...] = jnp.maximum(
        jnp.dot(w2t[...], h.astype(jnp.bfloat16),
                preferred_element_type=f32) + b2c[...],
        0.0).astype(jnp.bfloat16)


BE_F = 5120


def _edge_feat(x_row, w1c, b1c, w2t, b2c):
    return pl.pallas_call(
        _edge_feat_body,
        grid=(2 * N_EDGES // BE_F,),
        in_specs=[
            pl.BlockSpec((1, BE_F), lambda i: (0, i)),
            pl.BlockSpec((256, 1), lambda i: (0, 0)),
            pl.BlockSpec((256, 1), lambda i: (0, 0)),
            pl.BlockSpec((DIM, 256), lambda i: (0, 0)),
            pl.BlockSpec((DIM, 1), lambda i: (0, 0)),
        ],
        out_specs=pl.BlockSpec((DIM, BE_F), lambda i: (0, i)),
        out_shape=jax.ShapeDtypeStruct((DIM, 2 * N_EDGES), jnp.bfloat16),
        compiler_params=pltpu.CompilerParams(
            dimension_semantics=("parallel",)),
    )(x_row, w1c, b1c, w2t, b2c)


def _msg_body(xp, xc, fp, fc, w1ap, w1bp, w1cp, b1p, w2p, b2p,
              w1ac, w1bc, w1cc, b1c, w2c, b2c, o):
    bf = jnp.bfloat16
    xpb = xp[...][0].astype(bf)
    xcb = xc[...][0].astype(bf)
    dimnum = (((0,), (0,)), ((), ()))
    fpt = lax.dot_general(fp[...], w1cp[...], dimnum,
                          preferred_element_type=f32)
    fct = lax.dot_general(fc[...], w1cc[...], dimnum,
                          preferred_element_type=f32)
    hp = (jnp.dot(xcb, w1ap[...], preferred_element_type=f32)
          + jnp.dot(xpb, w1bp[...], preferred_element_type=f32)
          + fpt + b1p[...])
    hp = jnp.maximum(hp, 0.0)
    o[0] = jnp.maximum(
        jnp.dot(hp.astype(bf), w2p[...], preferred_element_type=f32)
        + b2p[...], 0.0)
    hc = (jnp.dot(xpb, w1ac[...], preferred_element_type=f32)
          + jnp.dot(xcb, w1bc[...], preferred_element_type=f32)
          + fct + b1c[...])
    hc = jnp.maximum(hc, 0.0)
    o[1] = jnp.maximum(
        jnp.dot(hc.astype(bf), w2c[...], preferred_element_type=f32)
        + b2c[...], 0.0)


def _msg_mlp(b_all, f_all, wp, wc):
    wspec = pl.BlockSpec((DIM, 256), lambda i: (0, 0))
    bspec = pl.BlockSpec((1, 256), lambda i: (0, 0))
    w2spec = pl.BlockSpec((256, DIM), lambda i: (0, 0))
    b2spec = pl.BlockSpec((1, DIM), lambda i: (0, 0))
    return pl.pallas_call(
        _msg_body,
        grid=(NBLK_SIDE,),
        in_specs=[
            pl.BlockSpec((1, BE, DIM), lambda i: (0, i, 0)),
            pl.BlockSpec((1, BE, DIM), lambda i: (1, i, 0)),
            pl.BlockSpec((DIM, BE), lambda i: (0, i)),
            pl.BlockSpec((DIM, BE), lambda i: (0, NBLK_SIDE + i)),
            wspec, wspec, wspec, bspec, w2spec, b2spec,
            wspec, wspec, wspec, bspec, w2spec, b2spec,
        ],
        out_specs=pl.BlockSpec((2, BE, DIM), lambda i: (0, i, 0)),
        out_shape=jax.ShapeDtypeStruct((2, N_EDGES, DIM), f32),
        compiler_params=pltpu.CompilerParams(
            dimension_semantics=("arbitrary",)),
    )(b_all, b_all, f_all, f_all, *wp, *wc)


def _combine_body(h_ref, sp, ip, mp, sc_, ic, mc, wa, wb, wc, b1, w2, b2, o):
    h = h_ref[...]
    s_p = sp[...][0] * ip[...][0] + mp[...]
    s_c = sc_[...][0] * ic[...][0] + mc[...]
    z = (jnp.dot(h, wa[...], preferred_element_type=f32)
         + jnp.dot(s_p, wb[...], preferred_element_type=f32)
         + jnp.dot(s_c, wc[...], preferred_element_type=f32)
         + b1[...])
    z = jnp.maximum(z, 0.0)
    z = jnp.maximum(jnp.dot(z, w2[...], preferred_element_type=f32)
                    + b2[...], 0.0)
    o[...] = jnp.maximum(h + z, 0.0)


def _combine(hidden, sums, inv, mask_p, mask_c, wa, wb, wc, b1, w2, b2):
    return pl.pallas_call(
        _combine_body,
        grid=(NBLK_N,),
        in_specs=[
            pl.BlockSpec((BN, DIM), lambda i: (i, 0)),
            pl.BlockSpec((1, BN, DIM), lambda i: (0, i, 0)),
            pl.BlockSpec((1, BN, 1), lambda i: (0, i, 0)),
            pl.BlockSpec((BN, DIM), lambda i: (i, 0)),
            pl.BlockSpec((1, BN, DIM), lambda i: (1, i, 0)),
            pl.BlockSpec((1, BN, 1), lambda i: (1, i, 0)),
            pl.BlockSpec((BN, DIM), lambda i: (i, 0)),
            pl.BlockSpec((DIM, 256), lambda i: (0, 0)),
            pl.BlockSpec((DIM, 256), lambda i: (0, 0)),
            pl.BlockSpec((DIM, 256), lambda i: (0, 0)),
            pl.BlockSpec((1, 256), lambda i: (0, 0)),
            pl.BlockSpec((256, DIM), lambda i: (0, 0)),
            pl.BlockSpec((1, DIM), lambda i: (0, 0)),
        ],
        out_specs=pl.BlockSpec((BN, DIM), lambda i: (i, 0)),
        out_shape=jax.ShapeDtypeStruct((N_NODES, DIM), f32),
        compiler_params=pltpu.CompilerParams(
            dimension_semantics=("parallel",)),
    )(hidden, sums, inv, mask_p, sums, inv, mask_c, wa, wb, wc, b1, w2, b2)


def kernel(batch_token, edge_p_node, edge_c_node, edge_p_indicate,
           edge_c_indicate, p_mask, c_mask, start_token, end_token,
           Wv1, bv1, Wv2, bv2, We1, be1, We2, be2, Wp1, bp1, Wp2, bp2,
           Wc1, bc1, Wc2, bc2, Wa1, ba1, Wa2, ba2):
    idx_all = jnp.concatenate([edge_p_node, edge_c_node]).astype(jnp.int32)
    idxg = idx_all.reshape(NW, NGRP_G, GB_G, CH)
    idxc = idx_all.reshape(NW, NCHUNK, CH)
    x_all = jnp.concatenate([edge_p_indicate,
                             edge_c_indicate]).reshape(1, -1)
    mask_p = p_mask[:, None] * start_token[None, :]
    mask_c = c_mask[:, None] * end_token[None, :]

    hidden = _node_mlp(batch_token, Wv1, bv1.reshape(1, -1),
                       Wv2, bv2.reshape(1, -1))
    f_all = _edge_feat(x_all, We1.reshape(256, 1), be1.reshape(256, 1),
                       We2.T.astype(jnp.bfloat16), be2.reshape(DIM, 1))

    bf16 = jnp.bfloat16
    wp = (Wp1[0:DIM].astype(bf16), Wp1[DIM:2 * DIM].astype(bf16),
          Wp1[2 * DIM:].astype(bf16), bp1.reshape(1, 256),
          Wp2.astype(bf16), bp2.reshape(1, DIM))
    wc = (Wc1[0:DIM].astype(bf16), Wc1[DIM:2 * DIM].astype(bf16),
          Wc1[2 * DIM:].astype(bf16), bc1.reshape(1, 256),
          Wc2.astype(bf16), bc2.reshape(1, DIM))

    wagg_a = Wa1[0:DIM]
    wagg_b = Wa1[DIM:2 * DIM]
    wagg_c = Wa1[2 * DIM:]

    cnts = _sc_count(idxc)
    inv = (1.0 / jnp.maximum(cnts[:, :, 0], 1.0)).reshape(2, N_NODES, 1)

    for _ in range(2):
        b_all = _sc_gather(hidden, idxg)
        s_all = _msg_mlp(b_all, f_all, wp, wc)
        sums = _sc_scatter(s_all, idxc)
        hidden = _combine(hidden, sums, inv, mask_p, mask_c,
                          wagg_a, wagg_b, wagg_c, ba1.reshape(1, -1),
                          Wa2, ba2.reshape(1, -1))
    return hidden
